# baseline (speedup 1.0000x reference)
import jax
import jax.numpy as jnp
from jax.experimental import pallas as pl

K = 1024
D = 256
NB = 4096
C = 128


def _vq_block(flat_ref, cb_ref, cbm2_ref, out_ref, loss_ref):
    flat = flat_ref[...]
    cb = cb_ref[...]
    cbm2 = cbm2_ref[...]
    f2 = jnp.sum(flat * flat, axis=1, keepdims=True)
    cb2 = jnp.sum(cb * cb, axis=1)
    mm = jax.lax.dot_general(flat, cbm2, (((1,), (1,)), ((), ())),
                             preferred_element_type=jnp.float32)
    iota_cf = jax.lax.broadcasted_iota(
        jnp.int32, (NB, C), 1).astype(jnp.float32)
    val = (f2 + cb2[0:C]) + mm[:, 0:C]
    ind = iota_cf
    for c in range(1, K // C):
        dc = (f2 + cb2[c * C:(c + 1) * C]) + mm[:, c * C:(c + 1) * C]
        lt = dc < val
        val = jnp.minimum(val, dc)
        ind = jnp.where(lt, iota_cf + float(c * C), ind)
    m = jnp.min(val, axis=1, keepdims=True)
    idxf = jnp.min(jnp.where(val == m, ind, float(K)), axis=1, keepdims=True)
    idx = idxf.astype(jnp.int32)
    iota = jax.lax.broadcasted_iota(jnp.int32, (NB, K), 1)
    oh = (iota == idx).astype(jnp.bfloat16)
    q = jax.lax.dot_general(oh, cb, (((1,), (0,)), ((), ())),
                            preferred_element_type=jnp.float32)
    out_ref[...] = flat + (q - flat)
    loss_ref[...] = jnp.full((1, 1, 128), jnp.sum(m), jnp.float32)


def kernel(latents, vq_weight, codebook):
    lat = jnp.transpose(latents, (0, 2, 3, 4, 1))
    lat_shape = lat.shape
    flat = lat.reshape(-1, D)
    n = flat.shape[0]
    nblk = n // NB
    out, lossp = pl.pallas_call(
        _vq_block,
        grid=(nblk,),
        in_specs=[pl.BlockSpec((NB, D), lambda i: (i, 0)),
                  pl.BlockSpec((K, D), lambda i: (0, 0)),
                  pl.BlockSpec((K, D), lambda i: (0, 0))],
        out_specs=[pl.BlockSpec((NB, D), lambda i: (i, 0)),
                   pl.BlockSpec((1, 1, 128), lambda i: (i, 0, 0))],
        out_shape=[jax.ShapeDtypeStruct((n, D), jnp.float32),
                   jax.ShapeDtypeStruct((nblk, 1, 128), jnp.float32)],
    )(flat, codebook, -2.0 * codebook)
    s = jnp.sum(lossp[:, 0, 0])
    mean = s / (n * D)
    vq_loss = mean * vq_weight + mean
    out5 = out.reshape(lat_shape)
    return jnp.transpose(out5, (0, 4, 1, 2, 3)), vq_loss
